# Initial kernel scaffold; baseline (speedup 1.0000x reference)
import jax
import jax.numpy as jnp
from jax.experimental import pallas as pl


def kernel(x, ln_g, ln_b, W1, b1, W2, b2):
    raise NotImplementedError("write your pallas kernel here")


# speedup vs baseline: 1.1927x; 1.1927x over previous
import functools

import jax
import jax.numpy as jnp
import numpy as np
from jax.experimental import pallas as pl
from jax.experimental.pallas import tpu as pltpu

H = 384
W = 384
NPH = 24
NPW = 24
PH = 16
PW = 16
NS = 2048
D = 96
DH = 24
NT = NPH * NPW * PH * PW
CT = 2304
RB = 8


def _score_body(x_ref, ln_g_ref, ln_b_ref, w1t_ref, b1_ref, w2p_ref, b2_ref,
                out_ref):
    xv = x_ref[...].reshape(RB * CT, D)
    mu = jnp.mean(xv, axis=-1, keepdims=True)
    var = jnp.var(xv, axis=-1, keepdims=True)
    xn = (xv - mu) / jnp.sqrt(var + 1e-5) * ln_g_ref[0] + ln_b_ref[0]
    h = jnp.dot(xn.astype(jnp.bfloat16), w1t_ref[:],
                preferred_element_type=jnp.float32) + b1_ref[0]
    h = 0.5 * h * (1.0 + jax.lax.erf(h * np.float32(np.sqrt(0.5))))
    imp = jnp.dot(h.astype(jnp.bfloat16), w2p_ref[:],
                  preferred_element_type=jnp.float32)[:, :1] + b2_ref[0]
    out_ref[...] = imp.reshape(RB, CT)


def _scores(x_tok, ln_g, ln_b, W1, b1, W2, b2):
    batch = x_tok.shape[0]
    rows = batch * NT // CT
    x_ch = x_tok.reshape(rows, CT, D)
    w1t = W1.T.astype(jnp.bfloat16)
    w2p = jnp.zeros((DH, 128), jnp.bfloat16).at[:, 0].set(
        W2[0].astype(jnp.bfloat16))
    out = pl.pallas_call(
        _score_body,
        grid=(rows // RB,),
        in_specs=[
            pl.BlockSpec((RB, CT, D), lambda g: (g, 0, 0)),
            pl.BlockSpec((1, D), lambda g: (0, 0)),
            pl.BlockSpec((1, D), lambda g: (0, 0)),
            pl.BlockSpec((D, DH), lambda g: (0, 0)),
            pl.BlockSpec((1, DH), lambda g: (0, 0)),
            pl.BlockSpec((DH, 128), lambda g: (0, 0)),
            pl.BlockSpec((1, 1), lambda g: (0, 0)),
        ],
        out_specs=pl.BlockSpec((RB, CT), lambda g: (g, 0)),
        out_shape=jax.ShapeDtypeStruct((rows, CT), jnp.float32),
    )(x_ch, ln_g.reshape(1, D), ln_b.reshape(1, D), w1t, b1.reshape(1, DH),
      w2p, b2.reshape(1, 1))
    return out.reshape(batch, NT)


def _grid_to_patch_idx(f):
    i = f // (PH * W)
    rem = f % (PH * W)
    a = rem // W
    rem2 = rem % W
    j = rem2 // PW
    bb = rem2 % PW
    return ((i * NPW + j) * PH + a) * PW + bb


def kernel(x, ln_g, ln_b, W1, b1, W2, b2):
    batch = x.shape[0]
    x_tok = x.reshape(batch, NT, D)
    scores = _scores(x_tok, ln_g, ln_b, W1, b1, W2, b2)
    sg = scores.reshape(batch, NPH, NPW, PH, PW)
    sg = jnp.transpose(sg, (0, 1, 3, 2, 4)).reshape(batch, NT)
    _, top_idx = jax.lax.top_k(sg, NS)
    t_idx = _grid_to_patch_idx(top_idx)
    return jnp.take_along_axis(x_tok, t_idx[:, :, None], axis=1)
